# baseline (speedup 1.0000x reference)
import functools

import jax
import jax.numpy as jnp
from jax import lax
from jax.experimental import pallas as pl
from jax.experimental.pallas import tpu as pltpu
from jax.experimental.pallas import tpu_sc as plsc

_N = 10000
_E = 160000
_D = 256
_NPAD = 10240
_RPT = 320
_TRASH = 320
_ACC_ROWS = 328
_PCH = 1024
_EPAD = 161792
_NPCH = _EPAD // _PCH
_CAP = 6400
_SUB = 64
_MBLK = 1024

_mesh = plsc.VectorSubcoreMesh(core_axis_name="c", subcore_axis_name="s")


@functools.partial(
    pl.kernel,
    mesh=_mesh,
    out_type=(
        jax.ShapeDtypeStruct((32, _CAP), jnp.int32),
        jax.ShapeDtypeStruct((32, _CAP), jnp.int32),
        jax.ShapeDtypeStruct((32, 384), jnp.float32),
        jax.ShapeDtypeStruct((32, 128), jnp.int32),
    ),
    scratch_types=[
        pltpu.VMEM((_PCH,), jnp.int32),
        pltpu.VMEM((_PCH,), jnp.int32),
        pltpu.VMEM((_PCH,), jnp.int32),
        pltpu.VMEM((_PCH,), jnp.int32),
        pltpu.VMEM((_CAP + 16,), jnp.int32),
        pltpu.VMEM((_CAP + 16,), jnp.int32),
        pltpu.VMEM((384,), jnp.float32),
        pltpu.VMEM((128,), jnp.int32),
        pltpu.SemaphoreType.DMA,
        pltpu.SemaphoreType.DMA,
    ],
    compiler_params=pltpu.CompilerParams(needs_layout_passes=False),
)
def _sc_partition(src_hbm, dst_hbm, bsrc_hbm, bdst_hbm, deg_hbm, cnt_hbm,
                  src_a, dst_a, src_b, dst_b, osrc_v, odst_v, deg_v, cnt_v,
                  sem_a, sem_b):
    c = lax.axis_index("c")
    s = lax.axis_index("s")
    w = c * 16 + s
    base = w * _RPT

    def prefill(i, carry):
        osrc_v[pl.ds(i * 16, 16)] = jnp.zeros((16,), jnp.int32)
        odst_v[pl.ds(i * 16, 16)] = jnp.full((16,), _TRASH, jnp.int32)
        return carry

    lax.fori_loop(0, _CAP // 16 + 1, prefill, 0)

    def zdeg(i, carry):
        deg_v[pl.ds(i * 16, 16)] = jnp.zeros((16,), jnp.float32)
        return carry

    lax.fori_loop(0, 384 // 16, zdeg, 0)

    ones16 = jnp.full((16,), 1.0, jnp.float32)

    def issue(t, sbuf, dbuf, sem):
        off = t * _PCH
        pltpu.async_copy(src_hbm.at[pl.ds(off, _PCH)], sbuf, sem)
        pltpu.async_copy(dst_hbm.at[pl.ds(off, _PCH)], dbuf, sem)

    def drain(sbuf, dbuf, sem):
        pltpu.make_async_copy(src_hbm.at[pl.ds(0, _PCH)], sbuf, sem).wait()
        pltpu.make_async_copy(dst_hbm.at[pl.ds(0, _PCH)], dbuf, sem).wait()

    lane15 = jnp.full((16,), 15, jnp.int32)
    lanes = lax.iota(jnp.int32, 16)

    def process(sbuf, dbuf, cnt):
        def group(g, cntv):
            vd = dbuf[pl.ds(g * 16, 16)]
            loc = vd - base
            msk = (loc >= 0) & (loc < _RPT)
            vs = sbuf[pl.ds(g * 16, 16)]
            mi = msk.astype(jnp.int32)
            csum = plsc.cumsum(mi)
            pos = jnp.where(msk, csum - 1 + cntv, _CAP + lanes)
            locc = jnp.clip(loc, 0, _RPT - 1)
            plsc.store_scatter(osrc_v, [pos], vs)
            plsc.store_scatter(odst_v, [pos], locc)
            plsc.addupdate_scatter(deg_v, [jnp.where(msk, locc, _RPT + lanes)],
                                   ones16)
            tot = jnp.take(csum, lane15)
            return jnp.minimum(cntv + tot, _CAP - 16)

        def group4(q, cntv):
            for u in range(4):
                cntv = group(q * 4 + u, cntv)
            return cntv

        return lax.fori_loop(0, _PCH // 64, group4, cnt)

    issue(0, src_a, dst_a, sem_a)
    issue(1, src_b, dst_b, sem_b)

    def pair(i, cnt):
        t0 = 2 * i
        drain(src_a, dst_a, sem_a)
        cnt = process(src_a, dst_a, cnt)

        @pl.when(t0 + 2 < _NPCH)
        def _():
            issue(t0 + 2, src_a, dst_a, sem_a)

        drain(src_b, dst_b, sem_b)
        cnt = process(src_b, dst_b, cnt)

        @pl.when(t0 + 3 < _NPCH)
        def _():
            issue(t0 + 3, src_b, dst_b, sem_b)

        return cnt

    cntv = lax.fori_loop(0, _NPCH // 2, pair, jnp.zeros((16,), jnp.int32))

    def wcnt(i, carry):
        cnt_v[pl.ds(i * 16, 16)] = cntv
        return carry

    lax.fori_loop(0, 8, wcnt, 0)

    pltpu.sync_copy(osrc_v.at[pl.ds(0, _CAP)], bsrc_hbm.at[w])
    pltpu.sync_copy(odst_v.at[pl.ds(0, _CAP)], bdst_hbm.at[w])
    pltpu.sync_copy(deg_v, deg_hbm.at[w])
    pltpu.sync_copy(cnt_v, cnt_hbm.at[w])


@functools.partial(
    pl.kernel,
    mesh=_mesh,
    out_type=jax.ShapeDtypeStruct((_NPAD, _D), jnp.float32),
    scratch_types=[
        pltpu.VMEM((_CAP,), jnp.int32),
        pltpu.VMEM((_CAP,), jnp.int32),
        pltpu.VMEM((_SUB, _D), jnp.float32),
        pltpu.VMEM((_SUB, _D), jnp.float32),
        pltpu.VMEM((128,), jnp.int32),
        pltpu.VMEM((_ACC_ROWS, _D), jnp.float32),
        pltpu.SemaphoreType.DMA,
        pltpu.SemaphoreType.DMA,
    ],
    compiler_params=pltpu.CompilerParams(needs_layout_passes=False),
)
def _sc_scatter(hs_hbm, bsrc_hbm, bdst_hbm, cnt_hbm, out_hbm,
                src_all, dst_all, rows_a, rows_b,
                cnt_v, acc_v, sem_a, sem_b):
    c = lax.axis_index("c")
    s = lax.axis_index("s")
    w = c * 16 + s
    base = w * _RPT
    pltpu.sync_copy(hs_hbm.at[pl.ds(base, _RPT)], acc_v.at[pl.ds(0, _RPT)])
    pltpu.sync_copy(bsrc_hbm.at[w], src_all)
    pltpu.sync_copy(bdst_hbm.at[w], dst_all)
    pltpu.sync_copy(cnt_hbm.at[w], cnt_v)
    cnt = cnt_v[pl.ds(0, 16)][0]
    npair = jnp.maximum((cnt + 127) // 128, 1)
    nsub = 2 * npair

    def issue(t, rbuf, sem):
        pltpu.async_copy(hs_hbm.at[src_all.at[pl.ds(t * _SUB, _SUB)]],
                         rbuf, sem)

    def drain(rbuf, sem):
        pltpu.make_async_copy(hs_hbm.at[src_all.at[pl.ds(0, _SUB)]],
                              rbuf, sem).wait()

    def process(t, rbuf):
        def group(g, c2):
            dvec = dst_all[pl.ds(t * _SUB + g * 16, 16)]
            for k in range(16):
                d = dvec[k]
                e = g * 16 + k
                vals = [rbuf[e, pl.ds(j * 16, 16)] for j in range(_D // 16)]
                for j in range(_D // 16):
                    plsc.addupdate(acc_v.at[d, pl.ds(j * 16, 16)], vals[j])
            return c2

        lax.fori_loop(0, _SUB // 16, group, 0)

    ring = ((rows_a, sem_a), (rows_b, sem_b))
    for r, (rbuf, sem) in enumerate(ring):
        issue(r, rbuf, sem)

    def pair(i, carry):
        t0 = 2 * i
        for r, (rbuf, sem) in enumerate(ring):
            drain(rbuf, sem)
            process(t0 + r, rbuf)

            @pl.when(t0 + r + 2 < nsub)
            def _():
                issue(t0 + r + 2, rbuf, sem)

        return carry

    lax.fori_loop(0, npair, pair, 0)
    pltpu.sync_copy(acc_v.at[pl.ds(0, _RPT)], out_hbm.at[pl.ds(base, _RPT)])


def _tc_mm_body(x_ref, w_ref, o_ref):
    o_ref[...] = jnp.dot(x_ref[...], w_ref[...],
                         preferred_element_type=jnp.float32)


def _tc_scale_body(m_ref, deg_ref, o_ref):
    o_ref[...] = lax.rsqrt(deg_ref[...] + 1.0) * m_ref[...]


def _tc_mid_body(s_ref, deg_ref, b_ref, w_ref, o_ref):
    dinv = lax.rsqrt(deg_ref[...] + 1.0)
    h = dinv * s_ref[...] + b_ref[...]
    h = h * jax.nn.sigmoid(h)
    o_ref[...] = dinv * jnp.dot(h, w_ref[...],
                                preferred_element_type=jnp.float32)


def _tc_final_body(s_ref, deg_ref, b_ref, wl_ref, bl_ref, o_ref):
    dinv = lax.rsqrt(deg_ref[...] + 1.0)
    h = dinv * s_ref[...] + b_ref[...]
    h = h * jax.nn.sigmoid(h)
    o_ref[...] = jnp.dot(h, wl_ref[...],
                         preferred_element_type=jnp.float32) + bl_ref[...]


def _row_block(i):
    return (i, 0)


def _rep_block(i):
    return (0, 0)


_tc_mm = pl.pallas_call(
    _tc_mm_body,
    grid=(_NPAD // _MBLK,),
    in_specs=[
        pl.BlockSpec((_MBLK, _D), _row_block),
        pl.BlockSpec((_D, _D), _rep_block),
    ],
    out_specs=pl.BlockSpec((_MBLK, _D), _row_block),
    out_shape=jax.ShapeDtypeStruct((_NPAD, _D), jnp.float32),
)

_tc_scale = pl.pallas_call(
    _tc_scale_body,
    grid=(_NPAD // _MBLK,),
    in_specs=[
        pl.BlockSpec((_MBLK, _D), _row_block),
        pl.BlockSpec((_MBLK, 1), _row_block),
    ],
    out_specs=pl.BlockSpec((_MBLK, _D), _row_block),
    out_shape=jax.ShapeDtypeStruct((_NPAD, _D), jnp.float32),
)

_tc_mid = pl.pallas_call(
    _tc_mid_body,
    grid=(_NPAD // _MBLK,),
    in_specs=[
        pl.BlockSpec((_MBLK, _D), _row_block),
        pl.BlockSpec((_MBLK, 1), _row_block),
        pl.BlockSpec((1, _D), _rep_block),
        pl.BlockSpec((_D, _D), _rep_block),
    ],
    out_specs=pl.BlockSpec((_MBLK, _D), _row_block),
    out_shape=jax.ShapeDtypeStruct((_NPAD, _D), jnp.float32),
)

_tc_final = pl.pallas_call(
    _tc_final_body,
    grid=(_NPAD // _MBLK,),
    in_specs=[
        pl.BlockSpec((_MBLK, _D), _row_block),
        pl.BlockSpec((_MBLK, 1), _row_block),
        pl.BlockSpec((1, _D), _rep_block),
        pl.BlockSpec((_D, _D), _rep_block),
        pl.BlockSpec((1, _D), _rep_block),
    ],
    out_specs=pl.BlockSpec((_MBLK, _D), _row_block),
    out_shape=jax.ShapeDtypeStruct((_NPAD, _D), jnp.float32),
)


def kernel(x, edge_index, W0, b0, W1, b1, Wl, bl):
    x_pad = jnp.pad(x, ((0, _NPAD - _N), (0, 0)))
    pad_e = _EPAD - _E
    src_pad = jnp.concatenate(
        [edge_index[0], jnp.zeros((pad_e,), jnp.int32)])
    dst_pad = jnp.concatenate(
        [edge_index[1], jnp.full((pad_e,), _NPAD, jnp.int32)])

    m0 = _tc_mm(x_pad, W0)
    bsrc, bdst, deg, cnt = _sc_partition(src_pad, dst_pad)
    deg_col = deg[:, :_RPT].reshape(_NPAD, 1)
    hs0 = _tc_scale(m0, deg_col)
    s0 = _sc_scatter(hs0, bsrc, bdst, cnt)
    hs1 = _tc_mid(s0, deg_col, b0.reshape(1, -1), W1)
    s1 = _sc_scatter(hs1, bsrc, bdst, cnt)
    out = _tc_final(s1, deg_col, b1.reshape(1, -1), Wl, bl.reshape(1, -1))
    return out[:_N]


# speedup vs baseline: 1.0060x; 1.0060x over previous
import functools

import jax
import jax.numpy as jnp
from jax import lax
from jax.experimental import pallas as pl
from jax.experimental.pallas import tpu as pltpu
from jax.experimental.pallas import tpu_sc as plsc

_N = 10000
_E = 160000
_D = 256
_NPAD = 10240
_RPT = 320
_TRASH = 320
_ACC_ROWS = 328
_PCH = 1024
_EPAD = 161792
_NPCH = _EPAD // _PCH
_CAP = 6400
_SUB = 64
_MBLK = 1024

_mesh = plsc.VectorSubcoreMesh(core_axis_name="c", subcore_axis_name="s")


@functools.partial(
    pl.kernel,
    mesh=_mesh,
    out_type=(
        jax.ShapeDtypeStruct((32, _CAP), jnp.int32),
        jax.ShapeDtypeStruct((32, _CAP), jnp.int32),
        jax.ShapeDtypeStruct((32, 384), jnp.float32),
        jax.ShapeDtypeStruct((32, 128), jnp.int32),
    ),
    scratch_types=[
        pltpu.VMEM((_PCH,), jnp.int32),
        pltpu.VMEM((_PCH,), jnp.int32),
        pltpu.VMEM((_PCH,), jnp.int32),
        pltpu.VMEM((_PCH,), jnp.int32),
        pltpu.VMEM((_CAP + 16,), jnp.int32),
        pltpu.VMEM((_CAP + 16,), jnp.int32),
        pltpu.VMEM((384,), jnp.float32),
        pltpu.VMEM((128,), jnp.int32),
        pltpu.SemaphoreType.DMA,
        pltpu.SemaphoreType.DMA,
    ],
    compiler_params=pltpu.CompilerParams(needs_layout_passes=False),
)
def _sc_partition(src_hbm, dst_hbm, bsrc_hbm, bdst_hbm, deg_hbm, cnt_hbm,
                  src_a, dst_a, src_b, dst_b, osrc_v, odst_v, deg_v, cnt_v,
                  sem_a, sem_b):
    c = lax.axis_index("c")
    s = lax.axis_index("s")
    w = c * 16 + s
    base = w * _RPT

    def prefill(i, carry):
        osrc_v[pl.ds(i * 16, 16)] = jnp.zeros((16,), jnp.int32)
        odst_v[pl.ds(i * 16, 16)] = jnp.full((16,), _TRASH, jnp.int32)
        return carry

    lax.fori_loop(0, _CAP // 16 + 1, prefill, 0)

    def zdeg(i, carry):
        deg_v[pl.ds(i * 16, 16)] = jnp.zeros((16,), jnp.float32)
        return carry

    lax.fori_loop(0, 384 // 16, zdeg, 0)

    ones16 = jnp.full((16,), 1.0, jnp.float32)

    def issue(t, sbuf, dbuf, sem):
        off = t * _PCH
        pltpu.async_copy(src_hbm.at[pl.ds(off, _PCH)], sbuf, sem)
        pltpu.async_copy(dst_hbm.at[pl.ds(off, _PCH)], dbuf, sem)

    def drain(sbuf, dbuf, sem):
        pltpu.make_async_copy(src_hbm.at[pl.ds(0, _PCH)], sbuf, sem).wait()
        pltpu.make_async_copy(dst_hbm.at[pl.ds(0, _PCH)], dbuf, sem).wait()

    lane15 = jnp.full((16,), 15, jnp.int32)
    lanes = lax.iota(jnp.int32, 16)

    def process(sbuf, dbuf, cnt):
        def group(g, cntv):
            vd = dbuf[pl.ds(g * 16, 16)]
            loc = vd - base
            msk = (loc >= 0) & (loc < _RPT)
            vs = sbuf[pl.ds(g * 16, 16)]
            mi = msk.astype(jnp.int32)
            csum = plsc.cumsum(mi)
            pos = jnp.where(msk, csum - 1 + cntv, _CAP + lanes)
            locc = jnp.clip(loc, 0, _RPT - 1)
            plsc.store_scatter(osrc_v, [pos], vs)
            plsc.store_scatter(odst_v, [pos], locc)
            plsc.addupdate_scatter(deg_v, [jnp.where(msk, locc, _RPT + lanes)],
                                   ones16)
            tot = jnp.take(csum, lane15)
            return jnp.minimum(cntv + tot, _CAP - 16)

        def group4(q, cntv):
            for u in range(4):
                cntv = group(q * 4 + u, cntv)
            return cntv

        return lax.fori_loop(0, _PCH // 64, group4, cnt)

    issue(0, src_a, dst_a, sem_a)
    issue(1, src_b, dst_b, sem_b)

    def pair(i, cnt):
        t0 = 2 * i
        drain(src_a, dst_a, sem_a)
        cnt = process(src_a, dst_a, cnt)

        @pl.when(t0 + 2 < _NPCH)
        def _():
            issue(t0 + 2, src_a, dst_a, sem_a)

        drain(src_b, dst_b, sem_b)
        cnt = process(src_b, dst_b, cnt)

        @pl.when(t0 + 3 < _NPCH)
        def _():
            issue(t0 + 3, src_b, dst_b, sem_b)

        return cnt

    cntv = lax.fori_loop(0, _NPCH // 2, pair, jnp.zeros((16,), jnp.int32))

    def wcnt(i, carry):
        cnt_v[pl.ds(i * 16, 16)] = cntv
        return carry

    lax.fori_loop(0, 8, wcnt, 0)

    pltpu.sync_copy(osrc_v.at[pl.ds(0, _CAP)], bsrc_hbm.at[w])
    pltpu.sync_copy(odst_v.at[pl.ds(0, _CAP)], bdst_hbm.at[w])
    pltpu.sync_copy(deg_v, deg_hbm.at[w])
    pltpu.sync_copy(cnt_v, cnt_hbm.at[w])


@functools.partial(
    pl.kernel,
    mesh=_mesh,
    out_type=jax.ShapeDtypeStruct((_NPAD, _D), jnp.float32),
    scratch_types=[
        pltpu.VMEM((_CAP,), jnp.int32),
        pltpu.VMEM((_CAP,), jnp.int32),
        pltpu.VMEM((_SUB, _D), jnp.float32),
        pltpu.VMEM((_SUB, _D), jnp.float32),
        pltpu.VMEM((128,), jnp.int32),
        pltpu.VMEM((_ACC_ROWS, _D), jnp.float32),
        pltpu.SemaphoreType.DMA,
        pltpu.SemaphoreType.DMA,
    ],
    compiler_params=pltpu.CompilerParams(needs_layout_passes=False),
)
def _sc_scatter(hs_hbm, bsrc_hbm, bdst_hbm, cnt_hbm, out_hbm,
                src_all, dst_all, rows_a, rows_b,
                cnt_v, acc_v, sem_a, sem_b):
    c = lax.axis_index("c")
    s = lax.axis_index("s")
    w = c * 16 + s
    base = w * _RPT
    pltpu.sync_copy(hs_hbm.at[pl.ds(base, _RPT)], acc_v.at[pl.ds(0, _RPT)])
    pltpu.sync_copy(bsrc_hbm.at[w], src_all)
    pltpu.sync_copy(bdst_hbm.at[w], dst_all)
    pltpu.sync_copy(cnt_hbm.at[w], cnt_v)
    cnt = cnt_v[pl.ds(0, 16)][0]
    npair = jnp.maximum((cnt + 127) // 128, 1)
    nsub = 2 * npair

    def issue(t, rbuf, sem):
        pltpu.async_copy(hs_hbm.at[src_all.at[pl.ds(t * _SUB, _SUB)]],
                         rbuf, sem)

    def drain(rbuf, sem):
        pltpu.make_async_copy(hs_hbm.at[src_all.at[pl.ds(0, _SUB)]],
                              rbuf, sem).wait()

    def process(t, rbuf):
        def group(g, c2):
            dvec = dst_all[pl.ds(t * _SUB + g * 16, 16)]
            for k in range(16):
                d = dvec[k]
                e = g * 16 + k
                vals = [rbuf[e, pl.ds(j * 16, 16)] for j in range(_D // 16)]
                for j in range(_D // 16):
                    plsc.addupdate(acc_v.at[d, pl.ds(j * 16, 16)], vals[j])
            return c2

        lax.fori_loop(0, _SUB // 16, group, 0)

    ring = ((rows_a, sem_a), (rows_b, sem_b))
    for r, (rbuf, sem) in enumerate(ring):
        issue(r, rbuf, sem)

    def pair(i, carry):
        t0 = 2 * i
        for r, (rbuf, sem) in enumerate(ring):
            drain(rbuf, sem)
            process(t0 + r, rbuf)

            @pl.when(t0 + r + 2 < nsub)
            def _():
                issue(t0 + r + 2, rbuf, sem)

        return carry

    lax.fori_loop(0, npair, pair, 0)
    pltpu.sync_copy(acc_v.at[pl.ds(0, _RPT)], out_hbm.at[pl.ds(base, _RPT)])


def _tc_first_body(x_ref, w_ref, deg_ref, o_ref):
    dinv = lax.rsqrt(deg_ref[...] + 1.0)
    o_ref[...] = dinv * jnp.dot(x_ref[...], w_ref[...],
                                preferred_element_type=jnp.float32)


def _tc_mid_body(s_ref, deg_ref, b_ref, w_ref, o_ref):
    dinv = lax.rsqrt(deg_ref[...] + 1.0)
    h = dinv * s_ref[...] + b_ref[...]
    h = h * jax.nn.sigmoid(h)
    o_ref[...] = dinv * jnp.dot(h, w_ref[...],
                                preferred_element_type=jnp.float32)


def _tc_final_body(s_ref, deg_ref, b_ref, wl_ref, bl_ref, o_ref):
    dinv = lax.rsqrt(deg_ref[...] + 1.0)
    h = dinv * s_ref[...] + b_ref[...]
    h = h * jax.nn.sigmoid(h)
    o_ref[...] = jnp.dot(h, wl_ref[...],
                         preferred_element_type=jnp.float32) + bl_ref[...]


def _row_block(i):
    return (i, 0)


def _rep_block(i):
    return (0, 0)


_tc_first = pl.pallas_call(
    _tc_first_body,
    grid=(_NPAD // _MBLK,),
    in_specs=[
        pl.BlockSpec((_MBLK, _D), _row_block),
        pl.BlockSpec((_D, _D), _rep_block),
        pl.BlockSpec((_MBLK, 1), _row_block),
    ],
    out_specs=pl.BlockSpec((_MBLK, _D), _row_block),
    out_shape=jax.ShapeDtypeStruct((_NPAD, _D), jnp.float32),
)

_tc_mid = pl.pallas_call(
    _tc_mid_body,
    grid=(_NPAD // _MBLK,),
    in_specs=[
        pl.BlockSpec((_MBLK, _D), _row_block),
        pl.BlockSpec((_MBLK, 1), _row_block),
        pl.BlockSpec((1, _D), _rep_block),
        pl.BlockSpec((_D, _D), _rep_block),
    ],
    out_specs=pl.BlockSpec((_MBLK, _D), _row_block),
    out_shape=jax.ShapeDtypeStruct((_NPAD, _D), jnp.float32),
)

_tc_final = pl.pallas_call(
    _tc_final_body,
    grid=(_NPAD // _MBLK,),
    in_specs=[
        pl.BlockSpec((_MBLK, _D), _row_block),
        pl.BlockSpec((_MBLK, 1), _row_block),
        pl.BlockSpec((1, _D), _rep_block),
        pl.BlockSpec((_D, _D), _rep_block),
        pl.BlockSpec((1, _D), _rep_block),
    ],
    out_specs=pl.BlockSpec((_MBLK, _D), _row_block),
    out_shape=jax.ShapeDtypeStruct((_NPAD, _D), jnp.float32),
)


def kernel(x, edge_index, W0, b0, W1, b1, Wl, bl):
    x_pad = jnp.pad(x, ((0, _NPAD - _N), (0, 0)))
    pad_e = _EPAD - _E
    src_pad = jnp.concatenate(
        [edge_index[0], jnp.zeros((pad_e,), jnp.int32)])
    dst_pad = jnp.concatenate(
        [edge_index[1], jnp.full((pad_e,), _NPAD, jnp.int32)])

    bsrc, bdst, deg, cnt = _sc_partition(src_pad, dst_pad)
    deg_col = deg[:, :_RPT].reshape(_NPAD, 1)
    hs0 = _tc_first(x_pad, W0, deg_col)
    s0 = _sc_scatter(hs0, bsrc, bdst, cnt)
    hs1 = _tc_mid(s0, deg_col, b0.reshape(1, -1), W1)
    s1 = _sc_scatter(hs1, bsrc, bdst, cnt)
    out = _tc_final(s1, deg_col, b1.reshape(1, -1), Wl, bl.reshape(1, -1))
    return out[:_N]
